# baseline (speedup 1.0000x reference)
import jax
import jax.numpy as jnp
from jax import lax
from jax.experimental import pallas as pl
from jax.experimental.pallas import tpu as pltpu

_N_E = 1024
_D = 256
_ROWS = 4608
_BLK = 1152
_GRID = _ROWS // _BLK


def _vq_tc_body(z_ref, w_ref, a_ref, b_ref, zq_ref, idx_ref, sse_ref):
    i = pl.program_id(0)
    zb = z_ref[...]
    w = w_ref[...]

    a = a_ref[...]
    b = b_ref[0, :]
    c = jnp.dot(zb, w.T, preferred_element_type=jnp.float32)
    sq = a + b[None, :] - 2.0 * c
    dist = jnp.sqrt(jnp.maximum(sq, 0.0))

    m = jnp.min(dist, axis=1, keepdims=True)
    lane = lax.broadcasted_iota(jnp.int32, dist.shape, 1)
    idx = jnp.min(jnp.where(dist == m, lane, _N_E), axis=1)
    idx_ref[...] = idx.reshape(1, 1, _BLK)

    onehot = (lane == idx[:, None]).astype(jnp.bfloat16)
    zq_ref[...] = jnp.dot(onehot, w.astype(jnp.bfloat16),
                          preferred_element_type=jnp.float32)

    @pl.when(i == 0)
    def _init():
        sse_ref[0, 0] = 0.0

    sse_ref[0, 0] += jnp.sum(m * m)


_vq_tc = pl.pallas_call(
    _vq_tc_body,
    grid=(_GRID,),
    in_specs=[
        pl.BlockSpec((_BLK, _D), lambda i: (i, 0)),
        pl.BlockSpec((_N_E, _D), lambda i: (0, 0)),
        pl.BlockSpec((_BLK, 1), lambda i: (i, 0)),
        pl.BlockSpec((1, _N_E), lambda i: (0, 0)),
    ],
    out_specs=[
        pl.BlockSpec((_BLK, _D), lambda i: (i, 0)),
        pl.BlockSpec((1, 1, _BLK), lambda i: (i, 0, 0)),
        pl.BlockSpec((1, 1), lambda i: (0, 0), memory_space=pltpu.SMEM),
    ],
    out_shape=[
        jax.ShapeDtypeStruct((_ROWS, _D), jnp.float32),
        jax.ShapeDtypeStruct((_GRID, 1, _BLK), jnp.int32),
        jax.ShapeDtypeStruct((1, 1), jnp.float32),
    ],
    compiler_params=pltpu.CompilerParams(
        dimension_semantics=("arbitrary",),
    ),
)


def kernel(z, W):
    input_shape = z.shape
    flat = z.reshape(_ROWS, _D)
    a = jnp.sum(flat ** 2, axis=1, keepdims=True)
    b = jnp.sum(W ** 2, axis=1).reshape(1, _N_E)

    z_q, idx3, sse = _vq_tc(flat, W, a, b)
    idx_flat = idx3.reshape(_ROWS)

    idx_out = idx_flat.reshape(input_shape[:-1])
    loss = sse[0, 0] * ((1.0 + 0.25) / (_ROWS * _D))
    return (z_q.reshape(input_shape), loss, idx_out)


# speedup vs baseline: 1.0129x; 1.0129x over previous
import jax
import jax.numpy as jnp
from jax import lax
from jax.experimental import pallas as pl
from jax.experimental.pallas import tpu as pltpu

_N_E = 1024
_D = 256
_ROWS = 4608
_BLK = 1152
_GRID = _ROWS // _BLK


def _vq_tc_body(z_ref, w_ref, wb_ref, a_ref, b_ref, zq_ref, idx_ref, sse_ref):
    i = pl.program_id(0)
    zb = z_ref[...]
    w = w_ref[...]

    a = a_ref[...]
    b = b_ref[0, :]
    c = jnp.dot(zb, w.T, preferred_element_type=jnp.float32)
    sq = a + b[None, :] - 2.0 * c
    dist = jnp.sqrt(jnp.maximum(sq, 0.0))

    m = jnp.min(dist, axis=1, keepdims=True)
    lane = lax.broadcasted_iota(jnp.int32, dist.shape, 1)
    idx = jnp.min(jnp.where(dist == m, lane, _N_E), axis=1)
    idx_ref[...] = idx.reshape(1, 1, _BLK)

    onehot = (lane == idx[:, None]).astype(jnp.bfloat16)
    zq_ref[...] = jnp.dot(onehot, wb_ref[...], preferred_element_type=jnp.float32)

    @pl.when(i == 0)
    def _init():
        sse_ref[0, 0] = 0.0

    sse_ref[0, 0] += jnp.sum(m * m)


_vq_tc = pl.pallas_call(
    _vq_tc_body,
    grid=(_GRID,),
    in_specs=[
        pl.BlockSpec((_BLK, _D), lambda i: (i, 0)),
        pl.BlockSpec((_N_E, _D), lambda i: (0, 0)),
        pl.BlockSpec((_N_E, _D), lambda i: (0, 0)),
        pl.BlockSpec((_BLK, 1), lambda i: (i, 0)),
        pl.BlockSpec((1, _N_E), lambda i: (0, 0)),
    ],
    out_specs=[
        pl.BlockSpec((_BLK, _D), lambda i: (i, 0)),
        pl.BlockSpec((1, 1, _BLK), lambda i: (i, 0, 0)),
        pl.BlockSpec((1, 1), lambda i: (0, 0), memory_space=pltpu.SMEM),
    ],
    out_shape=[
        jax.ShapeDtypeStruct((_ROWS, _D), jnp.float32),
        jax.ShapeDtypeStruct((_GRID, 1, _BLK), jnp.int32),
        jax.ShapeDtypeStruct((1, 1), jnp.float32),
    ],
    compiler_params=pltpu.CompilerParams(
        dimension_semantics=("arbitrary",),
    ),
)


def kernel(z, W):
    input_shape = z.shape
    flat = z.reshape(_ROWS, _D)
    a = jnp.sum(flat ** 2, axis=1, keepdims=True)
    b = jnp.sum(W ** 2, axis=1).reshape(1, _N_E)

    z_q, idx3, sse = _vq_tc(flat, W, W.astype(jnp.bfloat16), a, b)
    idx_flat = idx3.reshape(_ROWS)

    idx_out = idx_flat.reshape(input_shape[:-1])
    loss = sse[0, 0] * ((1.0 + 0.25) / (_ROWS * _D))
    return (z_q.reshape(input_shape), loss, idx_out)
